# Initial kernel scaffold; baseline (speedup 1.0000x reference)
import jax
import jax.numpy as jnp
from jax.experimental import pallas as pl


def kernel(x, edge_index, W1_l, b1, W1_r, W2_l, b2, W2_r):
    raise NotImplementedError("write your pallas kernel here")


# speedup vs baseline: 7.3801x; 7.3801x over previous
import functools

import jax
import jax.numpy as jnp
from jax import lax
from jax.experimental import pallas as pl
from jax.experimental.pallas import tpu as pltpu
from jax.experimental.pallas import tpu_sc as plsc

N_NODES = 10000
N_EDGES = 320000
D_IN = 128
D_HID = 256
D_OUT = 256

NC = 2
NS = 16
NW = NC * NS
BLK = 128
NBLK = 79
E_PAD = NW * NBLK * BLK
RPT = 632
ACC_N = NS * RPT

_sc_mesh = plsc.VectorSubcoreMesh(core_axis_name="c", subcore_axis_name="s")


@functools.partial(
    pl.kernel,
    mesh=_sc_mesh,
    out_type=[
        jax.ShapeDtypeStruct((NC, ACC_N, D_IN), jnp.float32),
        jax.ShapeDtypeStruct((NC * ACC_N,), jnp.float32),
    ],
    scratch_types=[
        pltpu.VMEM_SHARED((ACC_N, D_IN), jnp.float32),
        pltpu.VMEM_SHARED((ACC_N,), jnp.float32),
        pltpu.VMEM((NBLK, BLK), jnp.int32),
        pltpu.VMEM((NBLK, BLK), jnp.int32),
        pltpu.VMEM((BLK, D_IN), jnp.float32),
        pltpu.VMEM((BLK,), jnp.float32),
        pltpu.VMEM((RPT,), jnp.float32),
        pltpu.SemaphoreType.DMA,
    ],
)
def _sc_segsum1(x_hbm, src_hbm, dst_hbm, z2_hbm, z1_hbm, ones_hbm,
                sum_out, deg_out, acc_sh, deg_sh, src_v, dst_v, rows_v,
                ones_v, deg_v, sem):
    c = lax.axis_index("c")
    s = lax.axis_index("s")
    wid = c * NS + s
    pltpu.sync_copy(src_hbm.at[wid], src_v)
    pltpu.sync_copy(dst_hbm.at[wid], dst_v)
    pltpu.sync_copy(ones_hbm, ones_v)
    pltpu.sync_copy(z2_hbm.at[pl.ds(s * RPT, RPT)], acc_sh.at[pl.ds(s * RPT, RPT)])
    pltpu.sync_copy(z1_hbm.at[pl.ds(s * RPT, RPT)], deg_v)
    pltpu.sync_copy(deg_v, deg_sh.at[pl.ds(s * RPT, RPT)])
    plsc.subcore_barrier()

    def body(j, carry):
        pltpu.async_copy(x_hbm.at[src_v.at[j]], rows_v, sem).wait()
        pltpu.sync_copy(rows_v, acc_sh.at[dst_v.at[j]], add=True)
        pltpu.sync_copy(ones_v, deg_sh.at[dst_v.at[j]], add=True)
        return carry

    lax.fori_loop(0, NBLK, body, 0)
    plsc.subcore_barrier()
    pltpu.sync_copy(acc_sh.at[pl.ds(s * RPT, RPT)], sum_out.at[c, pl.ds(s * RPT, RPT)])
    pltpu.sync_copy(deg_sh.at[pl.ds(s * RPT, RPT)], deg_v)
    pltpu.sync_copy(deg_v, deg_out.at[pl.ds(c * ACC_N + s * RPT, RPT)])


@functools.partial(
    pl.kernel,
    mesh=_sc_mesh,
    out_type=jax.ShapeDtypeStruct((2, NC, ACC_N, D_IN), jnp.float32),
    scratch_types=[
        pltpu.VMEM_SHARED((ACC_N, D_IN), jnp.float32),
        pltpu.VMEM((NBLK, BLK), jnp.int32),
        pltpu.VMEM((NBLK, BLK), jnp.int32),
        pltpu.VMEM((BLK, D_IN), jnp.float32),
        pltpu.SemaphoreType.DMA,
    ],
)
def _sc_segsum2(hlo_hbm, hhi_hbm, src_hbm, dst_hbm, z2_hbm,
                sum_out, acc_sh, src_v, dst_v, rows_v, sem):
    c = lax.axis_index("c")
    s = lax.axis_index("s")
    wid = c * NS + s
    pltpu.sync_copy(src_hbm.at[wid], src_v)
    pltpu.sync_copy(dst_hbm.at[wid], dst_v)
    for half, tab in ((0, hlo_hbm), (1, hhi_hbm)):
        pltpu.sync_copy(z2_hbm.at[pl.ds(s * RPT, RPT)],
                        acc_sh.at[pl.ds(s * RPT, RPT)])
        plsc.subcore_barrier()

        def body(j, carry):
            pltpu.async_copy(tab.at[src_v.at[j]], rows_v, sem).wait()
            pltpu.sync_copy(rows_v, acc_sh.at[dst_v.at[j]], add=True)
            return carry

        lax.fori_loop(0, NBLK, body, 0)
        plsc.subcore_barrier()
        pltpu.sync_copy(acc_sh.at[pl.ds(s * RPT, RPT)],
                        sum_out.at[half, c, pl.ds(s * RPT, RPT)])
        plsc.subcore_barrier()


def _dense1_body(parts_ref, degp_ref, x_ref, wl_ref, wr_ref, b_ref,
                 hlo_ref, hhi_ref):
    summed = parts_ref[0] + parts_ref[1]
    deg = jnp.maximum(degp_ref[0] + degp_ref[1], 1.0)
    agg = summed * (1.0 / deg)
    dn = (((1,), (1,)), ((), ()))
    z = (lax.dot_general(agg, wl_ref[...], dn, preferred_element_type=jnp.float32)
         + lax.dot_general(x_ref[...], wr_ref[...], dn,
                           preferred_element_type=jnp.float32)
         + b_ref[...])
    h = jnp.maximum(z, 0.0)
    hlo_ref[...] = h[:, :D_IN]
    hhi_ref[...] = h[:, D_IN:]


def _dense2_body(parts_ref, degp_ref, hlo_ref, hhi_ref, wl_ref, wr_ref,
                 b_ref, out_ref):
    s_lo = parts_ref[0, 0] + parts_ref[0, 1]
    s_hi = parts_ref[1, 0] + parts_ref[1, 1]
    rdeg = 1.0 / jnp.maximum(degp_ref[0] + degp_ref[1], 1.0)
    dn = (((1,), (1,)), ((), ()))
    out_ref[...] = (
        lax.dot_general(s_lo * rdeg, wl_ref[:, :D_IN], dn,
                        preferred_element_type=jnp.float32)
        + lax.dot_general(s_hi * rdeg, wl_ref[:, D_IN:], dn,
                          preferred_element_type=jnp.float32)
        + lax.dot_general(hlo_ref[...], wr_ref[:, :D_IN], dn,
                          preferred_element_type=jnp.float32)
        + lax.dot_general(hhi_ref[...], wr_ref[:, D_IN:], dn,
                          preferred_element_type=jnp.float32)
        + b_ref[...])


_BR = 2000


def kernel(x, edge_index, W1_l, b1, W1_r, W2_l, b2, W2_r):
    src = edge_index[0].astype(jnp.int32)
    dst = edge_index[1].astype(jnp.int32)
    npad = E_PAD - N_EDGES
    pad_ids = jnp.arange(npad, dtype=jnp.int32)
    src3 = jnp.concatenate([src, pad_ids % N_NODES]).reshape(NW, NBLK, BLK)
    dst3 = jnp.concatenate(
        [dst, N_NODES + (pad_ids % (ACC_N - N_NODES))]).reshape(NW, NBLK, BLK)
    z2 = jnp.zeros((ACC_N, D_IN), jnp.float32)
    z1 = jnp.zeros((ACC_N,), jnp.float32)
    ones = jnp.ones((BLK,), jnp.float32)

    sum1, deg1 = _sc_segsum1(x, src3, dst3, z2, z1, ones)

    grid = (N_NODES // _BR,)
    h_lo, h_hi = pl.pallas_call(
        _dense1_body,
        grid=grid,
        in_specs=[
            pl.BlockSpec((NC, _BR, D_IN), lambda i: (0, i, 0)),
            pl.BlockSpec((NC, _BR, 1), lambda i: (0, i, 0)),
            pl.BlockSpec((_BR, D_IN), lambda i: (i, 0)),
            pl.BlockSpec((D_HID, D_IN), lambda i: (0, 0)),
            pl.BlockSpec((D_HID, D_IN), lambda i: (0, 0)),
            pl.BlockSpec((1, D_HID), lambda i: (0, 0)),
        ],
        out_specs=[
            pl.BlockSpec((_BR, D_IN), lambda i: (i, 0)),
            pl.BlockSpec((_BR, D_IN), lambda i: (i, 0)),
        ],
        out_shape=[
            jax.ShapeDtypeStruct((N_NODES, D_IN), jnp.float32),
            jax.ShapeDtypeStruct((N_NODES, D_IN), jnp.float32),
        ],
    )(sum1, deg1.reshape(NC, ACC_N, 1), x, W1_l, W1_r, b1.reshape(1, D_HID))

    sum2 = _sc_segsum2(h_lo, h_hi, src3, dst3, z2)

    out = pl.pallas_call(
        _dense2_body,
        grid=grid,
        in_specs=[
            pl.BlockSpec((2, NC, _BR, D_IN), lambda i: (0, 0, i, 0)),
            pl.BlockSpec((NC, _BR, 1), lambda i: (0, i, 0)),
            pl.BlockSpec((_BR, D_IN), lambda i: (i, 0)),
            pl.BlockSpec((_BR, D_IN), lambda i: (i, 0)),
            pl.BlockSpec((D_OUT, D_HID), lambda i: (0, 0)),
            pl.BlockSpec((D_OUT, D_HID), lambda i: (0, 0)),
            pl.BlockSpec((1, D_OUT), lambda i: (0, 0)),
        ],
        out_specs=pl.BlockSpec((_BR, D_OUT), lambda i: (i, 0)),
        out_shape=jax.ShapeDtypeStruct((N_NODES, D_OUT), jnp.float32),
    )(sum2, deg1.reshape(NC, ACC_N, 1), h_lo, h_hi, W2_l, W2_r,
      b2.reshape(1, D_OUT))
    return out


# speedup vs baseline: 11.1966x; 1.5171x over previous
import functools

import jax
import jax.numpy as jnp
from jax import lax
from jax.experimental import pallas as pl
from jax.experimental.pallas import tpu as pltpu
from jax.experimental.pallas import tpu_sc as plsc

N_NODES = 10000
N_EDGES = 320000
D_IN = 128
D_HID = 256
D_OUT = 256

NC = 2
NS = 16
NW = NC * NS
BLK = 128
CHK = 8
NCHK = 10
NBLK = CHK * NCHK
E_PAD = NW * NBLK * BLK
RPT = 632
ACC_N = NS * RPT

_sc_mesh = plsc.VectorSubcoreMesh(core_axis_name="c", subcore_axis_name="s")


@functools.partial(
    pl.kernel,
    mesh=_sc_mesh,
    out_type=[
        jax.ShapeDtypeStruct((NC, ACC_N, D_IN), jnp.float32),
        jax.ShapeDtypeStruct((NC * ACC_N,), jnp.float32),
    ],
    scratch_types=[
        pltpu.VMEM_SHARED((ACC_N, D_IN), jnp.float32),
        pltpu.VMEM_SHARED((ACC_N,), jnp.float32),
        pltpu.VMEM((2, CHK, BLK), jnp.int32),
        pltpu.VMEM((2, CHK, BLK), jnp.int32),
        pltpu.VMEM((2, BLK, D_IN), jnp.float32),
        pltpu.VMEM((BLK,), jnp.float32),
        pltpu.VMEM((RPT,), jnp.float32),
        pltpu.SemaphoreType.DMA,
        pltpu.SemaphoreType.DMA,
    ],
)
def _sc_segsum1(x_hbm, src_hbm, dst_hbm, z2_hbm, z1_hbm, ones_hbm,
                sum_out, deg_out, acc_sh, deg_sh, src_v, dst_v, rows_v,
                ones_v, deg_v, sem, isem):
    c = lax.axis_index("c")
    s = lax.axis_index("s")
    wid = c * NS + s
    pltpu.sync_copy(ones_hbm, ones_v)
    pltpu.sync_copy(src_hbm.at[wid, pl.ds(0, CHK)], src_v.at[0])
    pltpu.sync_copy(dst_hbm.at[wid, pl.ds(0, CHK)], dst_v.at[0])
    pltpu.sync_copy(z2_hbm.at[pl.ds(s * RPT, RPT)], acc_sh.at[pl.ds(s * RPT, RPT)])
    pltpu.sync_copy(z1_hbm.at[pl.ds(s * RPT, RPT)], deg_v)
    pltpu.sync_copy(deg_v, deg_sh.at[pl.ds(s * RPT, RPT)])
    plsc.subcore_barrier()

    pltpu.async_copy(x_hbm.at[src_v.at[0, 0]], rows_v.at[0], sem)

    def chunk(g, carry):
        par = g % 2

        @pl.when(g < NCHK - 1)
        def _():
            pltpu.async_copy(src_hbm.at[wid, pl.ds((g + 1) * CHK, CHK)],
                             src_v.at[(g + 1) % 2], isem)
            pltpu.async_copy(dst_hbm.at[wid, pl.ds((g + 1) * CHK, CHK)],
                             dst_v.at[(g + 1) % 2], isem)

        for b in range(CHK):
            if b < CHK - 1:
                pltpu.async_copy(x_hbm.at[src_v.at[par, b + 1]],
                                 rows_v.at[(b + 1) % 2], sem)
            else:
                @pl.when(g < NCHK - 1)
                def _():
                    pltpu.make_async_copy(
                        src_hbm.at[wid, pl.ds(0, CHK)], src_v.at[0], isem
                    ).wait()
                    pltpu.make_async_copy(
                        dst_hbm.at[wid, pl.ds(0, CHK)], dst_v.at[0], isem
                    ).wait()
                    pltpu.async_copy(x_hbm.at[src_v.at[(g + 1) % 2, 0]],
                                     rows_v.at[0], sem)
            pltpu.sync_copy(ones_v, deg_sh.at[dst_v.at[par, b]], add=True)
            pltpu.make_async_copy(x_hbm.at[src_v.at[par, b]],
                                  rows_v.at[b % 2], sem).wait()
            pltpu.sync_copy(rows_v.at[b % 2], acc_sh.at[dst_v.at[par, b]],
                            add=True)
        return carry

    lax.fori_loop(0, NCHK, chunk, 0)
    plsc.subcore_barrier()
    pltpu.sync_copy(acc_sh.at[pl.ds(s * RPT, RPT)], sum_out.at[c, pl.ds(s * RPT, RPT)])
    pltpu.sync_copy(deg_sh.at[pl.ds(s * RPT, RPT)], deg_v)
    pltpu.sync_copy(deg_v, deg_out.at[pl.ds(c * ACC_N + s * RPT, RPT)])


@functools.partial(
    pl.kernel,
    mesh=_sc_mesh,
    out_type=jax.ShapeDtypeStruct((2, NC, ACC_N, D_IN), jnp.float32),
    scratch_types=[
        pltpu.VMEM_SHARED((ACC_N, D_IN), jnp.float32),
        pltpu.VMEM((2, CHK, BLK), jnp.int32),
        pltpu.VMEM((2, CHK, BLK), jnp.int32),
        pltpu.VMEM((2, BLK, D_IN), jnp.float32),
        pltpu.SemaphoreType.DMA,
        pltpu.SemaphoreType.DMA,
    ],
)
def _sc_segsum2(hlo_hbm, hhi_hbm, src_hbm, dst_hbm, z2_hbm,
                sum_out, acc_sh, src_v, dst_v, rows_v, sem, isem):
    c = lax.axis_index("c")
    s = lax.axis_index("s")
    wid = c * NS + s
    for half, tab in ((0, hlo_hbm), (1, hhi_hbm)):
        pltpu.sync_copy(src_hbm.at[wid, pl.ds(0, CHK)], src_v.at[0])
        pltpu.sync_copy(dst_hbm.at[wid, pl.ds(0, CHK)], dst_v.at[0])
        pltpu.sync_copy(z2_hbm.at[pl.ds(s * RPT, RPT)],
                        acc_sh.at[pl.ds(s * RPT, RPT)])
        plsc.subcore_barrier()

        pltpu.async_copy(tab.at[src_v.at[0, 0]], rows_v.at[0], sem)

        def chunk(g, carry):
            par = g % 2

            @pl.when(g < NCHK - 1)
            def _():
                pltpu.async_copy(src_hbm.at[wid, pl.ds((g + 1) * CHK, CHK)],
                                 src_v.at[(g + 1) % 2], isem)
                pltpu.async_copy(dst_hbm.at[wid, pl.ds((g + 1) * CHK, CHK)],
                                 dst_v.at[(g + 1) % 2], isem)

            for b in range(CHK):
                if b < CHK - 1:
                    pltpu.async_copy(tab.at[src_v.at[par, b + 1]],
                                     rows_v.at[(b + 1) % 2], sem)
                else:
                    @pl.when(g < NCHK - 1)
                    def _():
                        pltpu.make_async_copy(
                            src_hbm.at[wid, pl.ds(0, CHK)], src_v.at[0], isem
                        ).wait()
                        pltpu.make_async_copy(
                            dst_hbm.at[wid, pl.ds(0, CHK)], dst_v.at[0], isem
                        ).wait()
                        pltpu.async_copy(tab.at[src_v.at[(g + 1) % 2, 0]],
                                         rows_v.at[0], sem)
                pltpu.make_async_copy(tab.at[src_v.at[par, b]],
                                      rows_v.at[b % 2], sem).wait()
                pltpu.sync_copy(rows_v.at[b % 2], acc_sh.at[dst_v.at[par, b]],
                                add=True)
            return carry

        lax.fori_loop(0, NCHK, chunk, 0)
        plsc.subcore_barrier()
        pltpu.sync_copy(acc_sh.at[pl.ds(s * RPT, RPT)],
                        sum_out.at[half, c, pl.ds(s * RPT, RPT)])
        plsc.subcore_barrier()


def _dense1_body(parts_ref, degp_ref, x_ref, wl_ref, wr_ref, b_ref,
                 hlo_ref, hhi_ref):
    summed = parts_ref[0] + parts_ref[1]
    deg = jnp.maximum(degp_ref[0] + degp_ref[1], 1.0)
    agg = summed * (1.0 / deg)
    dn = (((1,), (1,)), ((), ()))
    z = (lax.dot_general(agg, wl_ref[...], dn, preferred_element_type=jnp.float32)
         + lax.dot_general(x_ref[...], wr_ref[...], dn,
                           preferred_element_type=jnp.float32)
         + b_ref[...])
    h = jnp.maximum(z, 0.0)
    hlo_ref[...] = h[:, :D_IN]
    hhi_ref[...] = h[:, D_IN:]


def _dense2_body(parts_ref, degp_ref, hlo_ref, hhi_ref, wl_ref, wr_ref,
                 b_ref, out_ref):
    s_lo = parts_ref[0, 0] + parts_ref[0, 1]
    s_hi = parts_ref[1, 0] + parts_ref[1, 1]
    rdeg = 1.0 / jnp.maximum(degp_ref[0] + degp_ref[1], 1.0)
    dn = (((1,), (1,)), ((), ()))
    out_ref[...] = (
        lax.dot_general(s_lo * rdeg, wl_ref[:, :D_IN], dn,
                        preferred_element_type=jnp.float32)
        + lax.dot_general(s_hi * rdeg, wl_ref[:, D_IN:], dn,
                          preferred_element_type=jnp.float32)
        + lax.dot_general(hlo_ref[...], wr_ref[:, :D_IN], dn,
                          preferred_element_type=jnp.float32)
        + lax.dot_general(hhi_ref[...], wr_ref[:, D_IN:], dn,
                          preferred_element_type=jnp.float32)
        + b_ref[...])


_BR = 2000


def kernel(x, edge_index, W1_l, b1, W1_r, W2_l, b2, W2_r):
    src = edge_index[0].astype(jnp.int32)
    dst = edge_index[1].astype(jnp.int32)
    npad = E_PAD - N_EDGES
    pad_ids = jnp.arange(npad, dtype=jnp.int32)
    src3 = jnp.concatenate([src, pad_ids % N_NODES]).reshape(NW, NBLK, BLK)
    dst3 = jnp.concatenate(
        [dst, N_NODES + (pad_ids % (ACC_N - N_NODES))]).reshape(NW, NBLK, BLK)
    z2 = jnp.zeros((ACC_N, D_IN), jnp.float32)
    z1 = jnp.zeros((ACC_N,), jnp.float32)
    ones = jnp.ones((BLK,), jnp.float32)

    sum1, deg1 = _sc_segsum1(x, src3, dst3, z2, z1, ones)

    grid = (N_NODES // _BR,)
    h_lo, h_hi = pl.pallas_call(
        _dense1_body,
        grid=grid,
        in_specs=[
            pl.BlockSpec((NC, _BR, D_IN), lambda i: (0, i, 0)),
            pl.BlockSpec((NC, _BR, 1), lambda i: (0, i, 0)),
            pl.BlockSpec((_BR, D_IN), lambda i: (i, 0)),
            pl.BlockSpec((D_HID, D_IN), lambda i: (0, 0)),
            pl.BlockSpec((D_HID, D_IN), lambda i: (0, 0)),
            pl.BlockSpec((1, D_HID), lambda i: (0, 0)),
        ],
        out_specs=[
            pl.BlockSpec((_BR, D_IN), lambda i: (i, 0)),
            pl.BlockSpec((_BR, D_IN), lambda i: (i, 0)),
        ],
        out_shape=[
            jax.ShapeDtypeStruct((N_NODES, D_IN), jnp.float32),
            jax.ShapeDtypeStruct((N_NODES, D_IN), jnp.float32),
        ],
    )(sum1, deg1.reshape(NC, ACC_N, 1), x, W1_l, W1_r, b1.reshape(1, D_HID))

    sum2 = _sc_segsum2(h_lo, h_hi, src3, dst3, z2)

    out = pl.pallas_call(
        _dense2_body,
        grid=grid,
        in_specs=[
            pl.BlockSpec((2, NC, _BR, D_IN), lambda i: (0, 0, i, 0)),
            pl.BlockSpec((NC, _BR, 1), lambda i: (0, i, 0)),
            pl.BlockSpec((_BR, D_IN), lambda i: (i, 0)),
            pl.BlockSpec((_BR, D_IN), lambda i: (i, 0)),
            pl.BlockSpec((D_OUT, D_HID), lambda i: (0, 0)),
            pl.BlockSpec((D_OUT, D_HID), lambda i: (0, 0)),
            pl.BlockSpec((1, D_OUT), lambda i: (0, 0)),
        ],
        out_specs=pl.BlockSpec((_BR, D_OUT), lambda i: (i, 0)),
        out_shape=jax.ShapeDtypeStruct((N_NODES, D_OUT), jnp.float32),
    )(sum2, deg1.reshape(NC, ACC_N, 1), h_lo, h_hi, W2_l, W2_r,
      b2.reshape(1, D_OUT))
    return out


# speedup vs baseline: 11.2943x; 1.0087x over previous
import functools

import jax
import jax.numpy as jnp
from jax import lax
from jax.experimental import pallas as pl
from jax.experimental.pallas import tpu as pltpu
from jax.experimental.pallas import tpu_sc as plsc

N_NODES = 10000
N_EDGES = 320000
D_IN = 128
D_HID = 256
D_OUT = 256

NC = 2
NS = 16
NW = NC * NS
BLK = 128
CHK = 8
NCHK1 = 10
NCHK2 = 20
NBLK1 = CHK * NCHK1
NBLK2 = CHK * NCHK2
E_PAD = NW * NBLK1 * BLK
RPT = 632
ACC_N = NS * RPT

_sc_mesh = plsc.VectorSubcoreMesh(core_axis_name="c", subcore_axis_name="s")


def _segsum_loop(tab_hbm, idx_load, idx_wait, src_v, dst_v, rows_v, acc_sh,
                 sem, nchk, per_block=None):
    pltpu.async_copy(tab_hbm.at[src_v.at[0, 0]], rows_v.at[0], sem)

    def chunk(g, carry):
        par = g % 2

        @pl.when(g < nchk - 1)
        def _():
            idx_load(g + 1)

        for b in range(CHK):
            if b < CHK - 1:
                pltpu.async_copy(tab_hbm.at[src_v.at[par, b + 1]],
                                 rows_v.at[(b + 1) % 2], sem)
            else:
                @pl.when(g < nchk - 1)
                def _():
                    idx_wait()
                    pltpu.async_copy(tab_hbm.at[src_v.at[(g + 1) % 2, 0]],
                                     rows_v.at[0], sem)
            if per_block is not None:
                per_block(par, b)
            pltpu.make_async_copy(tab_hbm.at[src_v.at[par, b]],
                                  rows_v.at[b % 2], sem).wait()
            pltpu.sync_copy(rows_v.at[b % 2], acc_sh.at[dst_v.at[par, b]],
                            add=True)
        return carry

    lax.fori_loop(0, nchk, chunk, 0)


@functools.partial(
    pl.kernel,
    mesh=_sc_mesh,
    out_type=[
        jax.ShapeDtypeStruct((NC, ACC_N, D_IN), jnp.float32),
        jax.ShapeDtypeStruct((NC * ACC_N,), jnp.float32),
    ],
    scratch_types=[
        pltpu.VMEM_SHARED((ACC_N, D_IN), jnp.float32),
        pltpu.VMEM_SHARED((ACC_N,), jnp.float32),
        pltpu.VMEM((2, CHK, BLK), jnp.int32),
        pltpu.VMEM((2, CHK, BLK), jnp.int32),
        pltpu.VMEM((2, BLK, D_IN), jnp.float32),
        pltpu.VMEM((BLK,), jnp.float32),
        pltpu.VMEM((RPT,), jnp.float32),
        pltpu.SemaphoreType.DMA,
        pltpu.SemaphoreType.DMA,
    ],
)
def _sc_segsum1(x_hbm, src_hbm, dst_hbm, z2_hbm, z1_hbm, ones_hbm,
                sum_out, deg_out, acc_sh, deg_sh, src_v, dst_v, rows_v,
                ones_v, deg_v, sem, isem):
    c = lax.axis_index("c")
    s = lax.axis_index("s")
    wid = c * NS + s
    pltpu.sync_copy(ones_hbm, ones_v)
    pltpu.sync_copy(src_hbm.at[wid, pl.ds(0, CHK)], src_v.at[0])
    pltpu.sync_copy(dst_hbm.at[wid, pl.ds(0, CHK)], dst_v.at[0])
    pltpu.sync_copy(z2_hbm.at[pl.ds(s * RPT, RPT)], acc_sh.at[pl.ds(s * RPT, RPT)])
    pltpu.sync_copy(z1_hbm.at[pl.ds(s * RPT, RPT)], deg_v)
    pltpu.sync_copy(deg_v, deg_sh.at[pl.ds(s * RPT, RPT)])
    plsc.subcore_barrier()

    def idx_load(g):
        pltpu.async_copy(src_hbm.at[wid, pl.ds(g * CHK, CHK)],
                         src_v.at[g % 2], isem)
        pltpu.async_copy(dst_hbm.at[wid, pl.ds(g * CHK, CHK)],
                         dst_v.at[g % 2], isem)

    def idx_wait():
        pltpu.make_async_copy(src_hbm.at[wid, pl.ds(0, CHK)], src_v.at[0],
                              isem).wait()
        pltpu.make_async_copy(dst_hbm.at[wid, pl.ds(0, CHK)], dst_v.at[0],
                              isem).wait()

    def per_block(par, b):
        pltpu.sync_copy(ones_v, deg_sh.at[dst_v.at[par, b]], add=True)

    _segsum_loop(x_hbm, idx_load, idx_wait, src_v, dst_v, rows_v, acc_sh,
                 sem, NCHK1, per_block)
    plsc.subcore_barrier()
    pltpu.sync_copy(acc_sh.at[pl.ds(s * RPT, RPT)], sum_out.at[c, pl.ds(s * RPT, RPT)])
    pltpu.sync_copy(deg_sh.at[pl.ds(s * RPT, RPT)], deg_v)
    pltpu.sync_copy(deg_v, deg_out.at[pl.ds(c * ACC_N + s * RPT, RPT)])


@functools.partial(
    pl.kernel,
    mesh=_sc_mesh,
    out_type=jax.ShapeDtypeStruct((NC, ACC_N, D_IN), jnp.float32),
    scratch_types=[
        pltpu.VMEM_SHARED((ACC_N, D_IN), jnp.float32),
        pltpu.VMEM((2, CHK, BLK), jnp.int32),
        pltpu.VMEM((2, CHK, BLK), jnp.int32),
        pltpu.VMEM((2, BLK, D_IN), jnp.float32),
        pltpu.SemaphoreType.DMA,
        pltpu.SemaphoreType.DMA,
    ],
)
def _sc_segsum2(h2_hbm, src2_hbm, dst2_hbm, z2_hbm,
                sum_out, acc_sh, src_v, dst_v, rows_v, sem, isem):
    c = lax.axis_index("c")
    s = lax.axis_index("s")
    pltpu.sync_copy(src2_hbm.at[c, s, pl.ds(0, CHK)], src_v.at[0])
    pltpu.sync_copy(dst2_hbm.at[s, pl.ds(0, CHK)], dst_v.at[0])
    pltpu.sync_copy(z2_hbm.at[pl.ds(s * RPT, RPT)],
                    acc_sh.at[pl.ds(s * RPT, RPT)])
    plsc.subcore_barrier()

    def idx_load(g):
        pltpu.async_copy(src2_hbm.at[c, s, pl.ds(g * CHK, CHK)],
                         src_v.at[g % 2], isem)
        pltpu.async_copy(dst2_hbm.at[s, pl.ds(g * CHK, CHK)],
                         dst_v.at[g % 2], isem)

    def idx_wait():
        pltpu.make_async_copy(src2_hbm.at[c, s, pl.ds(0, CHK)], src_v.at[0],
                              isem).wait()
        pltpu.make_async_copy(dst2_hbm.at[s, pl.ds(0, CHK)], dst_v.at[0],
                              isem).wait()

    _segsum_loop(h2_hbm, idx_load, idx_wait, src_v, dst_v, rows_v, acc_sh,
                 sem, NCHK2)
    plsc.subcore_barrier()
    pltpu.sync_copy(acc_sh.at[pl.ds(s * RPT, RPT)],
                    sum_out.at[c, pl.ds(s * RPT, RPT)])


_DN = (((1,), (1,)), ((), ()))


def _dense_r_body(x_ref, wr_ref, b_ref, out_ref):
    out_ref[...] = lax.dot_general(
        x_ref[...], wr_ref[...], _DN,
        preferred_element_type=jnp.float32) + b_ref[...]


def _dense1_body(parts_ref, degp_ref, xr_ref, wl_ref, h_ref):
    summed = parts_ref[0] + parts_ref[1]
    deg = jnp.maximum(degp_ref[0] + degp_ref[1], 1.0)
    agg = summed * (1.0 / deg)
    z = lax.dot_general(agg, wl_ref[...], _DN,
                        preferred_element_type=jnp.float32) + xr_ref[...]
    h_ref[...] = jnp.maximum(z, 0.0)


def _dense2_body(parts_ref, degp_ref, hr_ref, wl_ref, out_ref):
    rdeg = 1.0 / jnp.maximum(degp_ref[0] + degp_ref[1], 1.0)
    out_ref[...] = (
        lax.dot_general(parts_ref[0] * rdeg, wl_ref[:, :D_IN], _DN,
                        preferred_element_type=jnp.float32)
        + lax.dot_general(parts_ref[1] * rdeg, wl_ref[:, D_IN:], _DN,
                          preferred_element_type=jnp.float32)
        + hr_ref[...])


_BR = 2000


def _full(shape):
    n = len(shape)
    return pl.BlockSpec(shape, lambda i: (0,) * n)


def kernel(x, edge_index, W1_l, b1, W1_r, W2_l, b2, W2_r):
    src = edge_index[0].astype(jnp.int32)
    dst = edge_index[1].astype(jnp.int32)
    npad = E_PAD - N_EDGES
    pad_ids = jnp.arange(npad, dtype=jnp.int32)
    src_p = jnp.concatenate([src, pad_ids % N_NODES])
    dst_p = jnp.concatenate([dst, N_NODES + (pad_ids % (ACC_N - N_NODES))])
    src3 = src_p.reshape(NW, NBLK1, BLK)
    dst3 = dst_p.reshape(NW, NBLK1, BLK)
    src2_3 = jnp.stack([2 * src_p, 2 * src_p + 1]).reshape(NC, NS, NBLK2, BLK)
    dst2_3 = dst_p.reshape(NS, NBLK2, BLK)
    z2 = jnp.zeros((ACC_N, D_IN), jnp.float32)
    z1 = jnp.zeros((ACC_N,), jnp.float32)
    ones = jnp.ones((BLK,), jnp.float32)

    grid = (N_NODES // _BR,)
    row_spec = pl.BlockSpec((_BR, D_IN), lambda i: (i, 0))
    hid_spec = pl.BlockSpec((_BR, D_HID), lambda i: (i, 0))
    parts_spec = pl.BlockSpec((NC, _BR, D_IN), lambda i: (0, i, 0))
    deg_spec = pl.BlockSpec((NC, _BR, 1), lambda i: (0, i, 0))

    sum1, deg1 = _sc_segsum1(x, src3, dst3, z2, z1, ones)
    xr = pl.pallas_call(
        _dense_r_body,
        grid=grid,
        in_specs=[row_spec, _full((D_HID, D_IN)), _full((1, D_HID))],
        out_specs=hid_spec,
        out_shape=jax.ShapeDtypeStruct((N_NODES, D_HID), jnp.float32),
    )(x, W1_r, b1.reshape(1, D_HID))

    deg3 = deg1.reshape(NC, ACC_N, 1)
    h = pl.pallas_call(
        _dense1_body,
        grid=grid,
        in_specs=[parts_spec, deg_spec, hid_spec, _full((D_HID, D_IN))],
        out_specs=hid_spec,
        out_shape=jax.ShapeDtypeStruct((N_NODES, D_HID), jnp.float32),
    )(sum1, deg3, xr, W1_l)

    sum2 = _sc_segsum2(h.reshape(2 * N_NODES, D_IN), src2_3, dst2_3, z2)
    hr = pl.pallas_call(
        _dense_r_body,
        grid=grid,
        in_specs=[hid_spec, _full((D_OUT, D_HID)), _full((1, D_OUT))],
        out_specs=pl.BlockSpec((_BR, D_OUT), lambda i: (i, 0)),
        out_shape=jax.ShapeDtypeStruct((N_NODES, D_OUT), jnp.float32),
    )(h, W2_r, b2.reshape(1, D_OUT))

    out = pl.pallas_call(
        _dense2_body,
        grid=grid,
        in_specs=[parts_spec, deg_spec,
                  pl.BlockSpec((_BR, D_OUT), lambda i: (i, 0)),
                  _full((D_OUT, D_HID))],
        out_specs=pl.BlockSpec((_BR, D_OUT), lambda i: (i, 0)),
        out_shape=jax.ShapeDtypeStruct((N_NODES, D_OUT), jnp.float32),
    )(sum2, deg3, hr, W2_l)
    return out


# speedup vs baseline: 11.4054x; 1.0098x over previous
import functools

import jax
import jax.numpy as jnp
import numpy as np
from jax import lax
from jax.experimental import pallas as pl
from jax.experimental.pallas import tpu as pltpu
from jax.experimental.pallas import tpu_sc as plsc

N_NODES = 10000
N_EDGES = 320000
D_IN = 128
D_HID = 256
D_OUT = 256

NC = 2
NS = 16
NW = NC * NS
BLK = 128
CHK = 8
NCHK1 = 10
NCHK2 = 20
NBLK1 = CHK * NCHK1
NBLK2 = CHK * NCHK2
E_PAD = NW * NBLK1 * BLK
RPT = 632
ACC_N = NS * RPT

_NPAD = E_PAD - N_EDGES
_PAD_SRC = jnp.asarray(np.arange(_NPAD, dtype=np.int32) % N_NODES)
_PAD_DST = jnp.asarray(
    N_NODES + np.arange(_NPAD, dtype=np.int32) % (ACC_N - N_NODES))

_sc_mesh = plsc.VectorSubcoreMesh(core_axis_name="c", subcore_axis_name="s")


def _segsum_loop(tab_hbm, idx_load, idx_wait, src_v, dst_v, rows_v, acc_sh,
                 sem, nchk, per_block=None):
    pltpu.async_copy(tab_hbm.at[src_v.at[0, 0]], rows_v.at[0], sem)

    def chunk(g, carry):
        par = g % 2

        @pl.when(g < nchk - 1)
        def _():
            idx_load(g + 1)

        for b in range(CHK):
            if b < CHK - 1:
                pltpu.async_copy(tab_hbm.at[src_v.at[par, b + 1]],
                                 rows_v.at[(b + 1) % 2], sem)
            else:
                @pl.when(g < nchk - 1)
                def _():
                    idx_wait()
                    pltpu.async_copy(tab_hbm.at[src_v.at[(g + 1) % 2, 0]],
                                     rows_v.at[0], sem)
            if per_block is not None:
                per_block(par, b)
            pltpu.make_async_copy(tab_hbm.at[src_v.at[par, b]],
                                  rows_v.at[b % 2], sem).wait()
            pltpu.sync_copy(rows_v.at[b % 2], acc_sh.at[dst_v.at[par, b]],
                            add=True)
        return carry

    lax.fori_loop(0, nchk, chunk, 0)


@functools.partial(
    pl.kernel,
    mesh=_sc_mesh,
    out_type=[
        jax.ShapeDtypeStruct((NC, ACC_N, D_IN), jnp.float32),
        jax.ShapeDtypeStruct((NC * ACC_N,), jnp.float32),
    ],
    scratch_types=[
        pltpu.VMEM_SHARED((ACC_N, D_IN), jnp.float32),
        pltpu.VMEM_SHARED((ACC_N,), jnp.float32),
        pltpu.VMEM((2, CHK, BLK), jnp.int32),
        pltpu.VMEM((2, CHK, BLK), jnp.int32),
        pltpu.VMEM((2, BLK, D_IN), jnp.float32),
        pltpu.VMEM((BLK,), jnp.float32),
        pltpu.VMEM((RPT,), jnp.float32),
        pltpu.SemaphoreType.DMA,
        pltpu.SemaphoreType.DMA,
    ],
)
def _sc_segsum1(x_hbm, src_hbm, dst_hbm, z2_hbm, z1_hbm, ones_hbm,
                sum_out, deg_out, acc_sh, deg_sh, src_v, dst_v, rows_v,
                ones_v, deg_v, sem, isem):
    c = lax.axis_index("c")
    s = lax.axis_index("s")
    wid = c * NS + s
    pltpu.sync_copy(ones_hbm, ones_v)
    pltpu.sync_copy(src_hbm.at[wid, pl.ds(0, CHK)], src_v.at[0])
    pltpu.sync_copy(dst_hbm.at[wid, pl.ds(0, CHK)], dst_v.at[0])
    pltpu.sync_copy(z2_hbm.at[pl.ds(s * RPT, RPT)], acc_sh.at[pl.ds(s * RPT, RPT)])
    pltpu.sync_copy(z1_hbm.at[pl.ds(s * RPT, RPT)], deg_v)
    pltpu.sync_copy(deg_v, deg_sh.at[pl.ds(s * RPT, RPT)])
    plsc.subcore_barrier()

    def idx_load(g):
        pltpu.async_copy(src_hbm.at[wid, pl.ds(g * CHK, CHK)],
                         src_v.at[g % 2], isem)
        pltpu.async_copy(dst_hbm.at[wid, pl.ds(g * CHK, CHK)],
                         dst_v.at[g % 2], isem)

    def idx_wait():
        pltpu.make_async_copy(src_hbm.at[wid, pl.ds(0, CHK)], src_v.at[0],
                              isem).wait()
        pltpu.make_async_copy(dst_hbm.at[wid, pl.ds(0, CHK)], dst_v.at[0],
                              isem).wait()

    def per_block(par, b):
        pltpu.sync_copy(ones_v, deg_sh.at[dst_v.at[par, b]], add=True)

    _segsum_loop(x_hbm, idx_load, idx_wait, src_v, dst_v, rows_v, acc_sh,
                 sem, NCHK1, per_block)
    plsc.subcore_barrier()
    pltpu.sync_copy(acc_sh.at[pl.ds(s * RPT, RPT)], sum_out.at[c, pl.ds(s * RPT, RPT)])
    pltpu.sync_copy(deg_sh.at[pl.ds(s * RPT, RPT)], deg_v)
    pltpu.sync_copy(deg_v, deg_out.at[pl.ds(c * ACC_N + s * RPT, RPT)])


@functools.partial(
    pl.kernel,
    mesh=_sc_mesh,
    out_type=jax.ShapeDtypeStruct((NC, ACC_N, D_IN), jnp.float32),
    scratch_types=[
        pltpu.VMEM_SHARED((ACC_N, D_IN), jnp.float32),
        pltpu.VMEM((2, CHK, BLK), jnp.int32),
        pltpu.VMEM((2, CHK, BLK), jnp.int32),
        pltpu.VMEM((2, BLK, D_IN), jnp.float32),
        pltpu.SemaphoreType.DMA,
        pltpu.SemaphoreType.DMA,
    ],
)
def _sc_segsum2(h2_hbm, src2_hbm, dst2_hbm, z2_hbm,
                sum_out, acc_sh, src_v, dst_v, rows_v, sem, isem):
    c = lax.axis_index("c")
    s = lax.axis_index("s")
    pltpu.sync_copy(src2_hbm.at[c, s, pl.ds(0, CHK)], src_v.at[0])
    pltpu.sync_copy(dst2_hbm.at[s, pl.ds(0, CHK)], dst_v.at[0])
    pltpu.sync_copy(z2_hbm.at[pl.ds(s * RPT, RPT)],
                    acc_sh.at[pl.ds(s * RPT, RPT)])
    plsc.subcore_barrier()

    def idx_load(g):
        pltpu.async_copy(src2_hbm.at[c, s, pl.ds(g * CHK, CHK)],
                         src_v.at[g % 2], isem)
        pltpu.async_copy(dst2_hbm.at[s, pl.ds(g * CHK, CHK)],
                         dst_v.at[g % 2], isem)

    def idx_wait():
        pltpu.make_async_copy(src2_hbm.at[c, s, pl.ds(0, CHK)], src_v.at[0],
                              isem).wait()
        pltpu.make_async_copy(dst2_hbm.at[s, pl.ds(0, CHK)], dst_v.at[0],
                              isem).wait()

    _segsum_loop(h2_hbm, idx_load, idx_wait, src_v, dst_v, rows_v, acc_sh,
                 sem, NCHK2)
    plsc.subcore_barrier()
    pltpu.sync_copy(acc_sh.at[pl.ds(s * RPT, RPT)],
                    sum_out.at[c, pl.ds(s * RPT, RPT)])


_DN = (((1,), (1,)), ((), ()))


def _dense_xr_body(x_ref, wr_ref, b_ref, out_ref):
    out_ref[...] = lax.dot_general(
        x_ref[...], wr_ref[...], _DN,
        preferred_element_type=jnp.float32) + b_ref[...]


def _dense1_body(parts_ref, degp_ref, xr_ref, wl_ref, h_ref):
    summed = parts_ref[0] + parts_ref[1]
    deg = jnp.maximum(degp_ref[0] + degp_ref[1], 1.0)
    agg = summed * (1.0 / deg)
    z = lax.dot_general(agg, wl_ref[...], _DN,
                        preferred_element_type=jnp.float32) + xr_ref[...]
    h_ref[...] = jnp.maximum(z, 0.0)


def _dense_hr_body(hlo_ref, hhi_ref, wr_ref, b_ref, out_ref):
    out_ref[...] = (
        lax.dot_general(hlo_ref[...], wr_ref[:, :D_IN], _DN,
                        preferred_element_type=jnp.float32)
        + lax.dot_general(hhi_ref[...], wr_ref[:, D_IN:], _DN,
                          preferred_element_type=jnp.float32)
        + b_ref[...])


def _dense2_body(parts_ref, degp_ref, hr_ref, wl_ref, out_ref):
    rdeg = 1.0 / jnp.maximum(degp_ref[0] + degp_ref[1], 1.0)
    out_ref[...] = (
        lax.dot_general(parts_ref[0] * rdeg, wl_ref[:, :D_IN], _DN,
                        preferred_element_type=jnp.float32)
        + lax.dot_general(parts_ref[1] * rdeg, wl_ref[:, D_IN:], _DN,
                          preferred_element_type=jnp.float32)
        + hr_ref[...])


_BR = 2000
_NRB = N_NODES // _BR


def _full(shape):
    n = len(shape)
    return pl.BlockSpec(shape, lambda *a: (0,) * n)


def kernel(x, edge_index, W1_l, b1, W1_r, W2_l, b2, W2_r):
    src = edge_index[0].astype(jnp.int32)
    dst = edge_index[1].astype(jnp.int32)
    src_p = jnp.concatenate([src, _PAD_SRC])
    dst_p = jnp.concatenate([dst, _PAD_DST])
    src3 = src_p.reshape(NW, NBLK1, BLK)
    dst3 = dst_p.reshape(NW, NBLK1, BLK)
    src2_3 = jnp.stack([src_p, src_p + N_NODES]).reshape(NC, NS, NBLK2, BLK)
    dst2_3 = dst_p.reshape(NS, NBLK2, BLK)
    z2 = jnp.zeros((ACC_N, D_IN), jnp.float32)
    z1 = jnp.zeros((ACC_N,), jnp.float32)
    ones = jnp.ones((BLK,), jnp.float32)

    grid = (_NRB,)
    row_spec = pl.BlockSpec((_BR, D_IN), lambda i: (i, 0))
    hid_spec = pl.BlockSpec((_BR, D_HID), lambda i: (i, 0))
    parts_spec = pl.BlockSpec((NC, _BR, D_IN), lambda i: (0, i, 0))
    deg_spec = pl.BlockSpec((NC, _BR, 1), lambda i: (0, i, 0))

    sum1, deg1 = _sc_segsum1(x, src3, dst3, z2, z1, ones)
    xr = pl.pallas_call(
        _dense_xr_body,
        grid=grid,
        in_specs=[row_spec, _full((D_HID, D_IN)), _full((1, D_HID))],
        out_specs=hid_spec,
        out_shape=jax.ShapeDtypeStruct((N_NODES, D_HID), jnp.float32),
    )(x, W1_r, b1.reshape(1, D_HID))

    deg3 = deg1.reshape(NC, ACC_N, 1)
    h2 = pl.pallas_call(
        _dense1_body,
        grid=(NC, _NRB),
        in_specs=[
            pl.BlockSpec((NC, _BR, D_IN), lambda f, i: (0, i, 0)),
            pl.BlockSpec((NC, _BR, 1), lambda f, i: (0, i, 0)),
            pl.BlockSpec((_BR, D_IN), lambda f, i: (i, f)),
            pl.BlockSpec((D_IN, D_IN), lambda f, i: (f, 0)),
        ],
        out_specs=pl.BlockSpec((_BR, D_IN), lambda f, i: (f * _NRB + i, 0)),
        out_shape=jax.ShapeDtypeStruct((2 * N_NODES, D_IN), jnp.float32),
    )(sum1, deg3, xr, W1_l)

    sum2 = _sc_segsum2(h2, src2_3, dst2_3, z2)
    hr = pl.pallas_call(
        _dense_hr_body,
        grid=grid,
        in_specs=[
            pl.BlockSpec((_BR, D_IN), lambda i: (i, 0)),
            pl.BlockSpec((_BR, D_IN), lambda i: (_NRB + i, 0)),
            _full((D_OUT, D_HID)),
            _full((1, D_OUT)),
        ],
        out_specs=pl.BlockSpec((_BR, D_OUT), lambda i: (i, 0)),
        out_shape=jax.ShapeDtypeStruct((N_NODES, D_OUT), jnp.float32),
    )(h2, h2, W2_r, b2.reshape(1, D_OUT))

    out = pl.pallas_call(
        _dense2_body,
        grid=grid,
        in_specs=[parts_spec, deg_spec,
                  pl.BlockSpec((_BR, D_OUT), lambda i: (i, 0)),
                  _full((D_OUT, D_HID))],
        out_specs=pl.BlockSpec((_BR, D_OUT), lambda i: (i, 0)),
        out_shape=jax.ShapeDtypeStruct((N_NODES, D_OUT), jnp.float32),
    )(sum2, deg3, hr, W2_l)
    return out
